# baseline (speedup 1.0000x reference)
import jax
import jax.numpy as jnp
from jax import lax
from jax.experimental import pallas as pl
from jax.experimental.pallas import tpu as pltpu
from jax.experimental.pallas import tpu_sc as plsc

N_NODES = 50000
HALF = 25000
FEAT = 64
NC, NS, LANES = 2, 16, 16
R_SC = 25088
ROWS_T = R_SC // NS
CH = 128
CPS = 8
SUP = CH * CPS
SUPS_T = 49
EDGES_T = SUP * SUPS_T
E_PAD = EDGES_T * NS
NCHUNK = E_PAD // CH
CHUNKS_T = NCHUNK // NS
BR = 2000
MDIV = 42800
MSH = 26
DUMLOC = ROWS_T
ACC_T = ROWS_T + 8
CAPC = NCHUNK + 8


def _sc_mesh():
    return plsc.VectorSubcoreMesh(core_axis_name="c", subcore_axis_name="s")


def _prep_body(src2d, dst2d, csrc, cdst, nsup, dsrc2, dloc2, nch2, deg16,
               src_v, dst_v, st_src, st_dst, ones_v, zero_v, loc2d, nv, sem,
               deg_sh):
    c = lax.axis_index("c")
    s = lax.axis_index("s")
    base = c * HALF
    r0 = s * ROWS_T
    cb = s * CHUNKS_T

    for i in range(CH):
        ones_v[i, :] = jnp.full((LANES,), 1.0, jnp.float32)
        zero_v[i, :] = jnp.zeros((LANES,), jnp.float32)
    for j in range(13):
        sz = CH if j < 12 else ROWS_T - 12 * CH
        pltpu.sync_copy(zero_v.at[pl.ds(0, sz)], deg_sh.at[pl.ds(r0 + j * CH, sz)])

    def flush_at(k):
        for j in range(8):
            pltpu.sync_copy(st_src.at[pl.ds(j * CH, CH)], csrc.at[c, cb + k * 8 + j])
            pltpu.sync_copy(st_dst.at[pl.ds(j * CH, CH)], cdst.at[c, cb + k * 8 + j])

    def shift_down():
        for i in range(64):
            st_src[pl.ds(i * LANES, LANES)] = st_src[pl.ds(SUP + i * LANES, LANES)]
            st_dst[pl.ds(i * LANES, LANES)] = st_dst[pl.ds(SUP + i * LANES, LANES)]

    lane = jax.lax.iota(jnp.int32, LANES)

    def super_body(u, carry):
        off, nfl = carry
        pltpu.sync_copy(src2d.at[pl.ds(cb + u * 8, 8)], src_v)
        pltpu.sync_copy(dst2d.at[pl.ds(cb + u * 8, 8)], dst_v)
        for i in range(64):
            row, col = i // 8, (i % 8) * LANES
            s16 = src_v[row, pl.ds(col, LANES)]
            d16 = dst_v[row, pl.ds(col, LANES)]
            m = (d16 >= base) & (d16 < base + HALF)
            run = plsc.cumsum(m.astype(jnp.int32))
            pos = off + run - 1
            plsc.store_scatter(st_src, [pos], s16, mask=m)
            plsc.store_scatter(st_dst, [pos], d16 - base, mask=m)
            off = off + jnp.max(run)
        full = off >= SUP

        @pl.when(full)
        def _():
            flush_at(nfl)
            shift_down()

        off = jnp.where(full, off - SUP, off)
        nfl = nfl + jnp.where(full, 1, 0)
        return off, nfl

    off, nfl = lax.fori_loop(0, CHUNKS_T // 8, super_body,
                             (jnp.int32(0), jnp.int32(0)))

    for i in range(64):
        idxs = off + i * LANES + lane
        plsc.store_scatter(st_src, [idxs], jnp.zeros((LANES,), jnp.int32))
        plsc.store_scatter(st_dst, [idxs], jnp.full((LANES,), HALF, jnp.int32))
    total = nfl + (off + SUP - 1) // SUP

    @pl.when(off > 0)
    def _():
        flush_at(nfl)

    @pl.when(off > SUP)
    def _():
        shift_down()
        flush_at(nfl + 1)

    nv[...] = jnp.zeros((LANES,), jnp.int32) + total
    pltpu.sync_copy(nv, nsup.at[c, s])

    plsc.subcore_barrier()

    def flush2_at(k):
        for j in range(8):
            pltpu.sync_copy(st_src.at[pl.ds(j * CH, CH)], dsrc2.at[c, s, k * 8 + j])
            pltpu.sync_copy(st_dst.at[pl.ds(j * CH, CH)], dloc2.at[c, s, k * 8 + j])

    def slab_body(v, carry):
        pltpu.sync_copy(nsup.at[c, v], nv)
        n_v = jnp.max(nv[...])

        def super2_body(u, carry2):
            off, nfl = carry2
            cb_v = v * CHUNKS_T + u * 8
            pltpu.sync_copy(csrc.at[c, pl.ds(cb_v, 8)], src_v)
            pltpu.sync_copy(cdst.at[c, pl.ds(cb_v, 8)], dst_v)
            for i in range(64):
                row, col = i // 8, (i % 8) * LANES
                s16 = src_v[row, pl.ds(col, LANES)]
                l16 = dst_v[row, pl.ds(col, LANES)]
                owner = (l16 * MDIV) >> MSH
                m = owner == s
                run = plsc.cumsum(m.astype(jnp.int32))
                pos = off + run - 1
                plsc.store_scatter(st_src, [pos], s16, mask=m)
                plsc.store_scatter(st_dst, [pos], l16 - s * ROWS_T, mask=m)
                off = off + jnp.max(run)
            full = off >= SUP

            @pl.when(full)
            def _():
                flush2_at(nfl)
                shift_down()

            off = jnp.where(full, off - SUP, off)
            nfl = nfl + jnp.where(full, 1, 0)
            return off, nfl

        return lax.fori_loop(0, n_v, super2_body, carry)

    off2, nfl2 = lax.fori_loop(0, NS, slab_body, (jnp.int32(0), jnp.int32(0)))

    for i in range(64):
        idxs = off2 + i * LANES + lane
        plsc.store_scatter(st_src, [idxs], jnp.zeros((LANES,), jnp.int32))
        plsc.store_scatter(st_dst, [idxs], jnp.full((LANES,), DUMLOC, jnp.int32))
    total2 = nfl2 + (off2 + SUP - 1) // SUP

    @pl.when(off2 > 0)
    def _():
        flush2_at(nfl2)

    @pl.when(off2 > SUP)
    def _():
        shift_down()
        flush2_at(nfl2 + 1)

    nv[...] = jnp.zeros((LANES,), jnp.int32) + total2 * 8
    pltpu.sync_copy(nv, nch2.at[c, s])


    def deg_body(u, carry):
        pltpu.sync_copy(cdst.at[c, pl.ds(cb + u * 8, 8)], loc2d)
        adds = [pltpu.async_copy(ones_v, deg_sh.at[loc2d.at[j]], sem, add=True)
                for j in range(8)]
        for a in adds:
            a.wait()
        return carry

    lax.fori_loop(0, total, deg_body, 0)
    plsc.subcore_barrier()
    pltpu.sync_copy(deg_sh.at[pl.ds(r0, ROWS_T)], deg16.at[c, pl.ds(r0, ROWS_T)])


def _prep(src2d, dst2d):
    return pl.kernel(
        _prep_body,
        out_type=[
            jax.ShapeDtypeStruct((NC, NCHUNK, CH), jnp.int32),
            jax.ShapeDtypeStruct((NC, NCHUNK, CH), jnp.int32),
            jax.ShapeDtypeStruct((NC, NS, LANES), jnp.int32),
            jax.ShapeDtypeStruct((NC, NS, CAPC, CH), jnp.int32),
            jax.ShapeDtypeStruct((NC, NS, CAPC, CH), jnp.int32),
            jax.ShapeDtypeStruct((NC, NS, LANES), jnp.int32),
            jax.ShapeDtypeStruct((NC, R_SC, LANES), jnp.float32),
        ],
        mesh=_sc_mesh(),
        compiler_params=pltpu.CompilerParams(use_tc_tiling_on_sc=False,
                                             needs_layout_passes=False),
        scratch_types=[
            pltpu.VMEM((8, CH), jnp.int32),
            pltpu.VMEM((8, CH), jnp.int32),
            pltpu.VMEM((3 * SUP, ), jnp.int32),
            pltpu.VMEM((3 * SUP, ), jnp.int32),
            pltpu.VMEM((CH, LANES), jnp.float32),
            pltpu.VMEM((CH, LANES), jnp.float32),
            pltpu.VMEM((8, CH), jnp.int32),
            pltpu.VMEM((LANES,), jnp.int32),
            pltpu.SemaphoreType.DMA,
            pltpu.VMEM_SHARED((R_SC, LANES), jnp.float32),
        ],
    )(src2d, dst2d)


def _layer_body(q, dsrc2, dloc2, nch2, sacc, sv, dv, rows, nv, sem_i, sem_g,
                acc):
    c = lax.axis_index("c")
    s = lax.axis_index("s")

    def zero_body(i, carry):
        for k in range(FEAT // LANES):
            acc[i, pl.ds(k * LANES, LANES)] = jnp.zeros((LANES,), jnp.float32)
        return carry

    lax.fori_loop(0, ACC_T, zero_body, 0)

    pltpu.sync_copy(nch2.at[c, s], nv)
    n = jnp.max(nv[...])

    @pl.when(n > 0)
    def _():
        pltpu.sync_copy(dsrc2.at[c, s, 0], sv.at[0])
        pltpu.sync_copy(dloc2.at[c, s, 0], dv.at[0])
        pltpu.async_copy(q.at[sv.at[0]], rows.at[0], sem_g)

    @pl.when(n > 1)
    def _():
        pltpu.sync_copy(dsrc2.at[c, s, 1], sv.at[1])
        pltpu.sync_copy(dloc2.at[c, s, 1], dv.at[1])

    col = [jax.lax.iota(jnp.int32, LANES) + k * LANES
           for k in range(FEAT // LANES)]

    def do_chunk(j, b):
        pltpu.make_async_copy(q.at[sv.at[b]], rows.at[b], sem_g).wait()

        b1 = (b + 1) % 3
        b2 = (b + 2) % 3

        @pl.when(j + 2 < n)
        def _():
            pltpu.async_copy(dsrc2.at[c, s, j + 2], sv.at[b2], sem_i)
            pltpu.async_copy(dloc2.at[c, s, j + 2], dv.at[b2], sem_i)

        @pl.when(j + 1 < n)
        def _():
            @pl.when(j >= 1)
            def _():
                pltpu.make_async_copy(dsrc2.at[c, s, j + 1], sv.at[b1],
                                      sem_i).wait()
                pltpu.make_async_copy(dloc2.at[c, s, j + 1], dv.at[b1],
                                      sem_i).wait()

            pltpu.async_copy(q.at[sv.at[b1]], rows.at[b1], sem_g)

        for g in range(CH // LANES):
            d16 = dv[b, pl.ds(g * LANES, LANES)]
            for r in range(LANES):
                tgt = d16.at[jnp.full((LANES,), r, jnp.int32)].get(
                    mode="promise_in_bounds")
                row = g * LANES + r
                for k in range(FEAT // LANES):
                    x = rows[b, row, pl.ds(k * LANES, LANES)]
                    plsc.addupdate_scatter(acc, [tgt, col[k]], x)

    def tri_body(p, carry):
        for b in range(3):
            j = 3 * p + b

            @pl.when(j < n)
            def _():
                do_chunk(j, b)
        return carry

    lax.fori_loop(0, (n + 2) // 3, tri_body, 0)
    pltpu.sync_copy(acc.at[pl.ds(0, ROWS_T)],
                    sacc.at[c, pl.ds(s * ROWS_T, ROWS_T)])


def _segsum(q, dsrc2, dloc2, nch2):
    return pl.kernel(
        _layer_body,
        out_type=jax.ShapeDtypeStruct((NC, R_SC, FEAT), jnp.float32),
        mesh=_sc_mesh(),
        compiler_params=pltpu.CompilerParams(use_tc_tiling_on_sc=False,
                                             needs_layout_passes=False),
        scratch_types=[
            pltpu.VMEM((3, CH), jnp.int32),
            pltpu.VMEM((3, CH), jnp.int32),
            pltpu.VMEM((3, CH, FEAT), jnp.float32),
            pltpu.VMEM((LANES,), jnp.int32),
            pltpu.SemaphoreType.DMA,
            pltpu.SemaphoreType.DMA,
            pltpu.VMEM((ACC_T, FEAT), jnp.float32),
        ],
    )(q, dsrc2, dloc2, nch2)


def _tc1_body(x_ref, w_ref, deg_ref, q_ref, dinv_ref):
    dinv = lax.rsqrt(deg_ref[...] + 1.0)
    p = jnp.dot(x_ref[...], w_ref[...], preferred_element_type=jnp.float32)
    q_ref[...] = dinv * p
    dinv_ref[...] = dinv


def _tc1(x, w1, deg):
    return pl.pallas_call(
        _tc1_body,
        grid=(N_NODES // BR,),
        in_specs=[
            pl.BlockSpec((BR, 128), lambda i: (i, 0)),
            pl.BlockSpec((128, FEAT), lambda i: (0, 0)),
            pl.BlockSpec((BR, 1), lambda i: (i, 0)),
        ],
        out_specs=[
            pl.BlockSpec((BR, FEAT), lambda i: (i, 0)),
            pl.BlockSpec((BR, 1), lambda i: (i, 0)),
        ],
        out_shape=[
            jax.ShapeDtypeStruct((N_NODES, FEAT), jnp.float32),
            jax.ShapeDtypeStruct((N_NODES, 1), jnp.float32),
        ],
    )(x, w1, deg)


def _tcmid_body(s_ref, q_ref, dinv_ref, b_ref, w_ref, qn_ref):
    dinv = dinv_ref[...]
    x = jnp.maximum(dinv * (s_ref[...] + q_ref[...]) + b_ref[...], 0.0)
    qn_ref[...] = dinv * jnp.dot(x, w_ref[...], preferred_element_type=jnp.float32)


def _tcmid(s, q, dinv, b, w):
    return pl.pallas_call(
        _tcmid_body,
        grid=(N_NODES // BR,),
        in_specs=[
            pl.BlockSpec((BR, FEAT), lambda i: (i, 0)),
            pl.BlockSpec((BR, FEAT), lambda i: (i, 0)),
            pl.BlockSpec((BR, 1), lambda i: (i, 0)),
            pl.BlockSpec((1, FEAT), lambda i: (0, 0)),
            pl.BlockSpec((FEAT, FEAT), lambda i: (0, 0)),
        ],
        out_specs=pl.BlockSpec((BR, FEAT), lambda i: (i, 0)),
        out_shape=jax.ShapeDtypeStruct((N_NODES, FEAT), jnp.float32),
    )(s, q, dinv, b, w)


def _tcfin_body(s_ref, q_ref, dinv_ref, b3_ref, wp_ref, bp_ref, wm1_ref,
                bm1_ref, wm2_ref, bm2_ref, out_ref, acc_ref):
    i = pl.program_id(0)
    x = jnp.maximum(dinv_ref[...] * (s_ref[...] + q_ref[...]) + b3_ref[...], 0.0)
    part = jnp.sum(x, axis=0, keepdims=True)

    @pl.when(i == 0)
    def _():
        acc_ref[...] = part

    @pl.when(i > 0)
    def _():
        acc_ref[...] += part

    @pl.when(i == pl.num_programs(0) - 1)
    def _():
        g = acc_ref[...] * (1.0 / N_NODES)
        cvec = jnp.dot(g, wp_ref[...], preferred_element_type=jnp.float32) + bp_ref[...]
        o = jnp.maximum(
            jnp.dot(cvec, wm1_ref[...], preferred_element_type=jnp.float32)
            + bm1_ref[...], 0.0)
        out_ref[...] = (
            jnp.dot(o, wm2_ref[...], preferred_element_type=jnp.float32)
            + bm2_ref[...])


def _tcfin(s, q, dinv, b3, wp, bp, wm1, bm1, wm2p, bm2p):
    return pl.pallas_call(
        _tcfin_body,
        grid=(N_NODES // BR,),
        in_specs=[
            pl.BlockSpec((BR, FEAT), lambda i: (i, 0)),
            pl.BlockSpec((BR, FEAT), lambda i: (i, 0)),
            pl.BlockSpec((BR, 1), lambda i: (i, 0)),
            pl.BlockSpec((1, FEAT), lambda i: (0, 0)),
            pl.BlockSpec((FEAT, 512), lambda i: (0, 0)),
            pl.BlockSpec((1, 512), lambda i: (0, 0)),
            pl.BlockSpec((512, 1024), lambda i: (0, 0)),
            pl.BlockSpec((1, 1024), lambda i: (0, 0)),
            pl.BlockSpec((1024, 256), lambda i: (0, 0)),
            pl.BlockSpec((1, 256), lambda i: (0, 0)),
        ],
        out_specs=pl.BlockSpec((1, 256), lambda i: (0, 0)),
        out_shape=jax.ShapeDtypeStruct((1, 256), jnp.float32),
        scratch_shapes=[pltpu.VMEM((1, FEAT), jnp.float32)],
    )(s, q, dinv, b3, wp, bp, wm1, bm1, wm2p, bm2p)


def kernel(x, edge_index, W1, b1, W2, b2, W3, b3, Wp, bp, Wm1, bm1, Wm2, bm2):
    e = edge_index.shape[1]
    pad = E_PAD - e
    src2d = jnp.concatenate(
        [edge_index[0], jnp.zeros((pad,), jnp.int32)]).reshape(NCHUNK, CH)
    dst2d = jnp.concatenate(
        [edge_index[1], jnp.full((pad,), N_NODES, jnp.int32)]).reshape(NCHUNK, CH)

    csrc, cdst, nsup, dsrc2, dloc2, nch2, deg16 = _prep(src2d, dst2d)
    deg = jnp.concatenate(
        [deg16[0, :HALF, 0], deg16[1, :HALF, 0]]).reshape(N_NODES, 1)

    q1, dinv = _tc1(x, W1, deg)
    s1 = _segsum(q1, dsrc2, dloc2, nch2)
    s1 = jnp.concatenate([s1[0, :HALF], s1[1, :HALF]], axis=0)
    q2 = _tcmid(s1, q1, dinv, b1.reshape(1, FEAT), W2)
    s2 = _segsum(q2, dsrc2, dloc2, nch2)
    s2 = jnp.concatenate([s2[0, :HALF], s2[1, :HALF]], axis=0)
    q3 = _tcmid(s2, q2, dinv, b2.reshape(1, FEAT), W3)
    s3 = _segsum(q3, dsrc2, dloc2, nch2)
    s3 = jnp.concatenate([s3[0, :HALF], s3[1, :HALF]], axis=0)

    wm2p = jnp.pad(Wm2, ((0, 0), (0, 56)))
    bm2p = jnp.pad(bm2, (0, 56))
    res = _tcfin(s3, q3, dinv, b3.reshape(1, FEAT), Wp, bp.reshape(1, 512),
                 Wm1, bm1.reshape(1, 1024), wm2p, bm2p.reshape(1, 256))
    p = 10
    return res[0, :p * p * 2].reshape(p, p, 2)


# speedup vs baseline: 2.2248x; 2.2248x over previous
import jax
import jax.numpy as jnp
from jax import lax
from jax.experimental import pallas as pl
from jax.experimental.pallas import tpu as pltpu
from jax.experimental.pallas import tpu_sc as plsc

N_NODES = 50000
HALF = 25000
FEAT = 64
NC, NS, LANES = 2, 16, 16
R_SC = 25088
ROWS_T = R_SC // NS
CH = 128
CPS = 8
SUP = CH * CPS
SUPS_T = 49
EDGES_T = SUP * SUPS_T
E_PAD = EDGES_T * NS
NCHUNK = E_PAD // CH
CHUNKS_T = NCHUNK // NS
BR = 2000


def _sc_mesh():
    return plsc.VectorSubcoreMesh(core_axis_name="c", subcore_axis_name="s")


def _prep_body(src2d, dst2d, csrc, cdst, nsup, deg16, src_v, dst_v, st_src,
               st_dst, ones_v, zero_v, loc2d, nv, sem, deg_sh):
    c = lax.axis_index("c")
    s = lax.axis_index("s")
    base = c * HALF
    r0 = s * ROWS_T
    cb = s * CHUNKS_T

    for i in range(CH):
        ones_v[i, :] = jnp.full((LANES,), 1.0, jnp.float32)
        zero_v[i, :] = jnp.zeros((LANES,), jnp.float32)
    for j in range(13):
        sz = CH if j < 12 else ROWS_T - 12 * CH
        pltpu.sync_copy(zero_v.at[pl.ds(0, sz)], deg_sh.at[pl.ds(r0 + j * CH, sz)])

    def flush_at(k):
        for j in range(8):
            pltpu.sync_copy(st_src.at[pl.ds(j * CH, CH)], csrc.at[c, cb + k * 8 + j])
            pltpu.sync_copy(st_dst.at[pl.ds(j * CH, CH)], cdst.at[c, cb + k * 8 + j])

    def shift_down():
        for i in range(64):
            st_src[pl.ds(i * LANES, LANES)] = st_src[pl.ds(SUP + i * LANES, LANES)]
            st_dst[pl.ds(i * LANES, LANES)] = st_dst[pl.ds(SUP + i * LANES, LANES)]

    lane = jax.lax.iota(jnp.int32, LANES)

    def super_body(u, carry):
        off, nfl = carry
        pltpu.sync_copy(src2d.at[pl.ds(cb + u * 8, 8)], src_v)
        pltpu.sync_copy(dst2d.at[pl.ds(cb + u * 8, 8)], dst_v)
        for i in range(64):
            row, col = i // 8, (i % 8) * LANES
            s16 = src_v[row, pl.ds(col, LANES)]
            d16 = dst_v[row, pl.ds(col, LANES)]
            m = (d16 >= base) & (d16 < base + HALF)
            run = plsc.cumsum(m.astype(jnp.int32))
            pos = off + run - 1
            plsc.store_scatter(st_src, [pos], s16, mask=m)
            plsc.store_scatter(st_dst, [pos], d16 - base, mask=m)
            off = off + plsc.all_reduce_population_count(m)
        full = jnp.max(off) >= SUP

        @pl.when(full)
        def _():
            flush_at(nfl)
            shift_down()

        off = jnp.where(full, off - SUP, off)
        nfl = nfl + jnp.where(full, 1, 0)
        return off, nfl

    off, nfl = lax.fori_loop(0, CHUNKS_T // 8, super_body,
                             (jnp.zeros((LANES,), jnp.int32), jnp.int32(0)))

    for i in range(64):
        idxs = off + i * LANES + lane
        plsc.store_scatter(st_src, [idxs], jnp.zeros((LANES,), jnp.int32))
        plsc.store_scatter(st_dst, [idxs], jnp.full((LANES,), HALF, jnp.int32))
    offs = jnp.max(off)
    total = nfl + (offs + SUP - 1) // SUP

    @pl.when(offs > 0)
    def _():
        flush_at(nfl)

    @pl.when(offs > SUP)
    def _():
        shift_down()
        flush_at(nfl + 1)

    nv[...] = jnp.zeros((LANES,), jnp.int32) + total
    pltpu.sync_copy(nv, nsup.at[c, s])

    plsc.subcore_barrier()

    def deg_body(u, carry):
        pltpu.sync_copy(cdst.at[c, pl.ds(cb + u * 8, 8)], loc2d)
        adds = [pltpu.async_copy(ones_v, deg_sh.at[loc2d.at[j]], sem, add=True)
                for j in range(8)]
        for a in adds:
            a.wait()
        return carry

    lax.fori_loop(0, total, deg_body, 0)
    plsc.subcore_barrier()
    pltpu.sync_copy(deg_sh.at[pl.ds(r0, ROWS_T)], deg16.at[c, pl.ds(r0, ROWS_T)])


def _prep(src2d, dst2d):
    return pl.kernel(
        _prep_body,
        out_type=[
            jax.ShapeDtypeStruct((NC, NCHUNK, CH), jnp.int32),
            jax.ShapeDtypeStruct((NC, NCHUNK, CH), jnp.int32),
            jax.ShapeDtypeStruct((NC, NS, LANES), jnp.int32),
            jax.ShapeDtypeStruct((NC, R_SC, LANES), jnp.float32),
        ],
        mesh=_sc_mesh(),
        compiler_params=pltpu.CompilerParams(use_tc_tiling_on_sc=False,
                                             needs_layout_passes=False),
        scratch_types=[
            pltpu.VMEM((8, CH), jnp.int32),
            pltpu.VMEM((8, CH), jnp.int32),
            pltpu.VMEM((3 * SUP, ), jnp.int32),
            pltpu.VMEM((3 * SUP, ), jnp.int32),
            pltpu.VMEM((CH, LANES), jnp.float32),
            pltpu.VMEM((CH, LANES), jnp.float32),
            pltpu.VMEM((8, CH), jnp.int32),
            pltpu.VMEM((LANES,), jnp.int32),
            pltpu.SemaphoreType.DMA,
            pltpu.VMEM_SHARED((R_SC, LANES), jnp.float32),
        ],
    )(src2d, dst2d)


def _layer_body(q, csrc, cdst, nsup, sacc, srcv, dlv, rows, nv, sem_i, sem_g,
                sem_s, acc_sh):
    c = lax.axis_index("c")
    s = lax.axis_index("s")
    r0 = s * ROWS_T
    cb = s * CHUNKS_T

    for t in range(3):
        for i in range(CH):
            for k in range(FEAT // LANES):
                rows[t, i, pl.ds(k * LANES, LANES)] = jnp.zeros((LANES,),
                                                                jnp.float32)
    for j in range(12):
        pltpu.sync_copy(rows.at[0], acc_sh.at[pl.ds(r0 + j * CH, CH)])
    pltpu.sync_copy(rows.at[0, pl.ds(0, ROWS_T - 12 * CH)],
                    acc_sh.at[pl.ds(r0 + 12 * CH, ROWS_T - 12 * CH)])
    plsc.subcore_barrier()

    pltpu.sync_copy(nsup.at[c, s], nv)
    n = jnp.max(nv[...])

    @pl.when(n > 0)
    def _():
        pltpu.sync_copy(csrc.at[c, pl.ds(cb, CPS)], srcv.at[0])
        pltpu.sync_copy(cdst.at[c, pl.ds(cb, CPS)], dlv.at[0])

    def do_super(u, b):
        @pl.when(u + 1 < n)
        def _():
            nb = cb + (u + 1) * CPS
            pltpu.async_copy(csrc.at[c, pl.ds(nb, CPS)], srcv.at[1 - b], sem_i)
            pltpu.async_copy(cdst.at[c, pl.ds(nb, CPS)], dlv.at[1 - b], sem_i)
        gd = [None] * CPS
        sd = [None] * CPS
        for j in range(3):
            gd[j] = pltpu.async_copy(q.at[srcv.at[b, j]], rows.at[j], sem_g)
        for j in range(CPS):
            gd[j].wait()
            sd[j] = pltpu.async_copy(rows.at[j % 3],
                                     acc_sh.at[dlv.at[b, j]], sem_s, add=True)
            if j + 3 < CPS:
                sd[j].wait()
                gd[j + 3] = pltpu.async_copy(q.at[srcv.at[b, j + 3]],
                                             rows.at[j % 3], sem_g)
        for j in range(CPS - 3, CPS):
            sd[j].wait()
        @pl.when(u + 1 < n)
        def _():
            nb = cb + (u + 1) * CPS
            pltpu.make_async_copy(csrc.at[c, pl.ds(nb, CPS)],
                                  srcv.at[1 - b], sem_i).wait()
            pltpu.make_async_copy(cdst.at[c, pl.ds(nb, CPS)],
                                  dlv.at[1 - b], sem_i).wait()

    def pair_body(p, carry):
        for b in range(2):
            u = 2 * p + b

            @pl.when(u < n)
            def _():
                do_super(u, b)
        return carry

    lax.fori_loop(0, (n + 1) // 2, pair_body, 0)
    plsc.subcore_barrier()
    pltpu.sync_copy(acc_sh.at[pl.ds(r0, ROWS_T)], sacc.at[c, pl.ds(r0, ROWS_T)])


def _segsum(q, csrc, cdst, nsup):
    return pl.kernel(
        _layer_body,
        out_type=jax.ShapeDtypeStruct((NC, R_SC, FEAT), jnp.float32),
        mesh=_sc_mesh(),
        compiler_params=pltpu.CompilerParams(use_tc_tiling_on_sc=False,
                                             needs_layout_passes=False),
        scratch_types=[
            pltpu.VMEM((2, CPS, CH), jnp.int32),
            pltpu.VMEM((2, CPS, CH), jnp.int32),
            pltpu.VMEM((3, CH, FEAT), jnp.float32),
            pltpu.VMEM((LANES,), jnp.int32),
            pltpu.SemaphoreType.DMA,
            pltpu.SemaphoreType.DMA,
            pltpu.SemaphoreType.DMA,
            pltpu.VMEM_SHARED((R_SC, FEAT), jnp.float32),
        ],
    )(q, csrc, cdst, nsup)


def _tc1_body(x_ref, w_ref, deg_ref, q_ref, dinv_ref):
    dinv = lax.rsqrt(deg_ref[...] + 1.0)
    p = jnp.dot(x_ref[...], w_ref[...], preferred_element_type=jnp.float32)
    q_ref[...] = dinv * p
    dinv_ref[...] = dinv


def _tc1(x, w1, deg):
    return pl.pallas_call(
        _tc1_body,
        grid=(N_NODES // BR,),
        in_specs=[
            pl.BlockSpec((BR, 128), lambda i: (i, 0)),
            pl.BlockSpec((128, FEAT), lambda i: (0, 0)),
            pl.BlockSpec((BR, 1), lambda i: (i, 0)),
        ],
        out_specs=[
            pl.BlockSpec((BR, FEAT), lambda i: (i, 0)),
            pl.BlockSpec((BR, 1), lambda i: (i, 0)),
        ],
        out_shape=[
            jax.ShapeDtypeStruct((N_NODES, FEAT), jnp.float32),
            jax.ShapeDtypeStruct((N_NODES, 1), jnp.float32),
        ],
    )(x, w1, deg)


def _tcmid_body(s_ref, q_ref, dinv_ref, b_ref, w_ref, qn_ref):
    dinv = dinv_ref[...]
    x = jnp.maximum(dinv * (s_ref[...] + q_ref[...]) + b_ref[...], 0.0)
    qn_ref[...] = dinv * jnp.dot(x, w_ref[...], preferred_element_type=jnp.float32)


def _tcmid(s, q, dinv, b, w):
    return pl.pallas_call(
        _tcmid_body,
        grid=(N_NODES // BR,),
        in_specs=[
            pl.BlockSpec((BR, FEAT), lambda i: (i, 0)),
            pl.BlockSpec((BR, FEAT), lambda i: (i, 0)),
            pl.BlockSpec((BR, 1), lambda i: (i, 0)),
            pl.BlockSpec((1, FEAT), lambda i: (0, 0)),
            pl.BlockSpec((FEAT, FEAT), lambda i: (0, 0)),
        ],
        out_specs=pl.BlockSpec((BR, FEAT), lambda i: (i, 0)),
        out_shape=jax.ShapeDtypeStruct((N_NODES, FEAT), jnp.float32),
    )(s, q, dinv, b, w)


def _tcfin_body(s_ref, q_ref, dinv_ref, b3_ref, wp_ref, bp_ref, wm1_ref,
                bm1_ref, wm2_ref, bm2_ref, out_ref, acc_ref):
    i = pl.program_id(0)
    x = jnp.maximum(dinv_ref[...] * (s_ref[...] + q_ref[...]) + b3_ref[...], 0.0)
    part = jnp.sum(x, axis=0, keepdims=True)

    @pl.when(i == 0)
    def _():
        acc_ref[...] = part

    @pl.when(i > 0)
    def _():
        acc_ref[...] += part

    @pl.when(i == pl.num_programs(0) - 1)
    def _():
        g = acc_ref[...] * (1.0 / N_NODES)
        cvec = jnp.dot(g, wp_ref[...], preferred_element_type=jnp.float32) + bp_ref[...]
        o = jnp.maximum(
            jnp.dot(cvec, wm1_ref[...], preferred_element_type=jnp.float32)
            + bm1_ref[...], 0.0)
        out_ref[...] = (
            jnp.dot(o, wm2_ref[...], preferred_element_type=jnp.float32)
            + bm2_ref[...])


def _tcfin(s, q, dinv, b3, wp, bp, wm1, bm1, wm2p, bm2p):
    return pl.pallas_call(
        _tcfin_body,
        grid=(N_NODES // BR,),
        in_specs=[
            pl.BlockSpec((BR, FEAT), lambda i: (i, 0)),
            pl.BlockSpec((BR, FEAT), lambda i: (i, 0)),
            pl.BlockSpec((BR, 1), lambda i: (i, 0)),
            pl.BlockSpec((1, FEAT), lambda i: (0, 0)),
            pl.BlockSpec((FEAT, 512), lambda i: (0, 0)),
            pl.BlockSpec((1, 512), lambda i: (0, 0)),
            pl.BlockSpec((512, 1024), lambda i: (0, 0)),
            pl.BlockSpec((1, 1024), lambda i: (0, 0)),
            pl.BlockSpec((1024, 256), lambda i: (0, 0)),
            pl.BlockSpec((1, 256), lambda i: (0, 0)),
        ],
        out_specs=pl.BlockSpec((1, 256), lambda i: (0, 0)),
        out_shape=jax.ShapeDtypeStruct((1, 256), jnp.float32),
        scratch_shapes=[pltpu.VMEM((1, FEAT), jnp.float32)],
    )(s, q, dinv, b3, wp, bp, wm1, bm1, wm2p, bm2p)


def kernel(x, edge_index, W1, b1, W2, b2, W3, b3, Wp, bp, Wm1, bm1, Wm2, bm2):
    e = edge_index.shape[1]
    pad = E_PAD - e
    src2d = jnp.concatenate(
        [edge_index[0], jnp.zeros((pad,), jnp.int32)]).reshape(NCHUNK, CH)
    dst2d = jnp.concatenate(
        [edge_index[1], jnp.full((pad,), N_NODES, jnp.int32)]).reshape(NCHUNK, CH)

    csrc, cdst, nsup, deg16 = _prep(src2d, dst2d)
    deg = jnp.concatenate(
        [deg16[0, :HALF, 0], deg16[1, :HALF, 0]]).reshape(N_NODES, 1)

    q1, dinv = _tc1(x, W1, deg)
    s1 = _segsum(q1, csrc, cdst, nsup)
    s1 = jnp.concatenate([s1[0, :HALF], s1[1, :HALF]], axis=0)
    q2 = _tcmid(s1, q1, dinv, b1.reshape(1, FEAT), W2)
    s2 = _segsum(q2, csrc, cdst, nsup)
    s2 = jnp.concatenate([s2[0, :HALF], s2[1, :HALF]], axis=0)
    q3 = _tcmid(s2, q2, dinv, b2.reshape(1, FEAT), W3)
    s3 = _segsum(q3, csrc, cdst, nsup)
    s3 = jnp.concatenate([s3[0, :HALF], s3[1, :HALF]], axis=0)

    wm2p = jnp.pad(Wm2, ((0, 0), (0, 56)))
    bm2p = jnp.pad(bm2, (0, 56))
    res = _tcfin(s3, q3, dinv, b3.reshape(1, FEAT), Wp, bp.reshape(1, 512),
                 Wm1, bm1.reshape(1, 1024), wm2p, bm2p.reshape(1, 256))
    p = 10
    return res[0, :p * p * 2].reshape(p, p, 2)
